# baseline (speedup 1.0000x reference)
import functools

import jax
import jax.numpy as jnp
import numpy as np
from jax import lax
from jax.experimental import pallas as pl
from jax.experimental.pallas import tpu as pltpu
from jax.experimental.pallas import tpu_sc as plsc

_PROJ_H = 12
_PROJ_W = 12
_N_ACTIONS = 96
_NCELL = _PROJ_H * _PROJ_W
_ORIGINS = {0: (0, 4), 1: (4, 2), 2: (4, 6), 3: (8, 0), 4: (8, 4), 5: (8, 8)}


def _build_proj_index():
    idx = []
    for g in range(_N_ACTIONS):
        b = g // 16
        loc = g % 16
        r, c = (loc // 4, loc % 4)
        br, bc = _ORIGINS[b]
        idx.append((br + r) * _PROJ_W + (bc + c))
    return np.asarray(idx, dtype=np.int32)

_PROJ_IDX_NP = _build_proj_index()
_SRC = [None] * _NCELL
for _g, _p in enumerate(_PROJ_IDX_NP):
    _SRC[int(_p)] = _g
_VALID_CELLS = [p for p in range(_NCELL) if _SRC[p] is not None]

_NC = 2
_NS = 16
_NW = _NC * _NS
_BCHUNK = 128
_LGRP = _BCHUNK // 16
_NSLAB = 3 * _PROJ_H


def _sc_body(obs_hbm, out_hbm, in_v0, in_v1, out_v0, out_v1, p2_v,
             semi0, semi1, semo0, semo1, semp2, *, nchunks):
    wid = lax.axis_index("s") * _NC + lax.axis_index("c")
    base = wid * (nchunks * _BCHUNK)

    zero = jnp.zeros((16,), jnp.float32)
    one = jnp.ones((16,), jnp.float32)
    ins = [in_v0, in_v1]
    outs = [out_v0, out_v1]
    semi = [semi0, semi1]
    semo = [semo0, semo1]

    def in_slice(ci):
        return obs_hbm.at[:, pl.ds(base + ci * _BCHUNK, _BCHUNK)]

    in_d = [None, None]
    in_d[0] = pltpu.async_copy(in_slice(0), ins[0], semi[0])

    @plsc.parallel_loop(0, _LGRP)
    def _(l):
        col = l * 16
        for p in range(_NCELL):
            if _SRC[p] is None:
                for buf in outs:
                    buf[p, pl.ds(col, 16)] = zero
                    buf[_NCELL + p, pl.ds(col, 16)] = zero
                p2_v[p, pl.ds(col, 16)] = zero
            else:
                p2_v[p, pl.ds(col, 16)] = one

    def run_compute(in_buf, out_buf):
        @plsc.parallel_loop(0, _LGRP, unroll=4)
        def _(l):
            col = l * 16
            for p in _VALID_CELLS:
                g = _SRC[p]
                x = in_buf[g, pl.ds(col, 16)]
                out_buf[p, pl.ds(col, 16)] = jnp.where(x > 0.5, one, zero)
                out_buf[_NCELL + p, pl.ds(col, 16)] = jnp.where(
                    x < -0.5, one, zero)

    out_d = [[], []]
    p2_d = []
    for ci in range(nchunks):
        pb = ci & 1
        qb = (ci + 1) & 1
        if ci + 1 < nchunks:
            in_d[qb] = pltpu.async_copy(in_slice(ci + 1), ins[qb], semi[qb])
        in_d[pb].wait()
        for d in out_d[pb]:
            d.wait()
        for d in p2_d:
            d.wait()
        p2_d = []
        run_compute(ins[pb], outs[pb])
        bcol = base + ci * _BCHUNK
        out_d[pb] = [
            pltpu.async_copy(outs[pb].at[pl.ds(p0 * _PROJ_W, _PROJ_W)],
                             out_hbm.at[s, :, pl.ds(bcol, _BCHUNK)], semo[pb])
            for s, p0 in [(h, h) for h in range(_PROJ_H)]
            + [(12 + h, 12 + h) for h in range(_PROJ_H)]
        ]
        p2_d = [
            pltpu.async_copy(p2_v.at[pl.ds(h * _PROJ_W, _PROJ_W)],
                             out_hbm.at[24 + h, :, pl.ds(bcol, _BCHUNK)],
                             semp2)
            for h in range(_PROJ_H)
        ]

    for ds in out_d:
        for d in ds:
            d.wait()
    for d in p2_d:
        d.wait()


@jax.jit
def kernel(observation):
    if observation.ndim == 1:
        observation = observation[None, :]
    bsz = observation.shape[0]

    step = _NW * _BCHUNK
    bpad = ((bsz + step - 1) // step) * step
    obs_t = observation.astype(jnp.float32).T
    if bpad != bsz:
        obs_t = jnp.pad(obs_t, ((0, 0), (0, bpad - bsz)))
    nchunks = bpad // step

    run = pl.kernel(
        functools.partial(_sc_body, nchunks=nchunks),
        out_type=jax.ShapeDtypeStruct((_NSLAB, _PROJ_W, bpad), jnp.float32),
        mesh=plsc.VectorSubcoreMesh(core_axis_name="c", subcore_axis_name="s"),
        compiler_params=pltpu.CompilerParams(
            needs_layout_passes=False, use_tc_tiling_on_sc=True,
            skip_device_barrier=True, disable_bounds_checks=True,
            disable_semaphore_checks=True),
        scratch_types=[
            pltpu.VMEM((_N_ACTIONS, _BCHUNK), jnp.float32),
            pltpu.VMEM((_N_ACTIONS, _BCHUNK), jnp.float32),
            pltpu.VMEM((2 * _NCELL, _BCHUNK), jnp.float32),
            pltpu.VMEM((2 * _NCELL, _BCHUNK), jnp.float32),
            pltpu.VMEM((_NCELL, _BCHUNK), jnp.float32),
            pltpu.SemaphoreType.DMA,
            pltpu.SemaphoreType.DMA,
            pltpu.SemaphoreType.DMA,
            pltpu.SemaphoreType.DMA,
            pltpu.SemaphoreType.DMA,
        ],
    )
    out3 = run(obs_t)
    board = jnp.transpose(out3.reshape(3, _PROJ_H, _PROJ_W, bpad),
                          (3, 0, 1, 2))[:bsz]
    return board.astype(observation.dtype)


# speedup vs baseline: 1.1422x; 1.1422x over previous
import functools

import jax
import jax.numpy as jnp
import numpy as np
from jax import lax
from jax.experimental import pallas as pl
from jax.experimental.pallas import tpu as pltpu
from jax.experimental.pallas import tpu_sc as plsc

_PROJ_H = 12
_PROJ_W = 12
_N_ACTIONS = 96
_NCELL = _PROJ_H * _PROJ_W
_ORIGINS = {0: (0, 4), 1: (4, 2), 2: (4, 6), 3: (8, 0), 4: (8, 4), 5: (8, 8)}


def _build_proj_index():
    idx = []
    for g in range(_N_ACTIONS):
        b = g // 16
        loc = g % 16
        r, c = (loc // 4, loc % 4)
        br, bc = _ORIGINS[b]
        idx.append((br + r) * _PROJ_W + (bc + c))
    return np.asarray(idx, dtype=np.int32)

_PROJ_IDX_NP = _build_proj_index()
_SRC = [None] * _NCELL
for _g, _p in enumerate(_PROJ_IDX_NP):
    _SRC[int(_p)] = _g
_VALID_CELLS = [p for p in range(_NCELL) if _SRC[p] is not None]

_NC = 2
_NS = 16
_NW = _NC * _NS
_BCHUNK = 128
_LGRP = _BCHUNK // 16
_NSLAB = 3 * _PROJ_H


def _sc_body(obs_hbm, out_hbm, in_v0, in_v1, out_v0, out_v1, p2_v,
             semi0, semi1, semo0, semo1, semp0, semp1, *, nchunks):
    wid = lax.axis_index("s") * _NC + lax.axis_index("c")
    base = wid * (nchunks * _BCHUNK)

    zero = jnp.zeros((16,), jnp.float32)
    one = jnp.ones((16,), jnp.float32)
    ins = [in_v0, in_v1]
    outs = [out_v0, out_v1]
    semi = [semi0, semi1]
    semo = [semo0, semo1]
    semp = [semp0, semp1]

    def in_slice(ci):
        return obs_hbm.at[:, pl.ds(base + ci * _BCHUNK, _BCHUNK)]

    in_d = [None, None]
    in_d[0] = pltpu.async_copy(in_slice(0), ins[0], semi[0])

    @plsc.parallel_loop(0, _LGRP)
    def _(l):
        col = l * 16
        for p in range(_NCELL):
            if _SRC[p] is None:
                for buf in outs:
                    buf[p, pl.ds(col, 16)] = zero
                    buf[_NCELL + p, pl.ds(col, 16)] = zero
                p2_v[p, pl.ds(col, 16)] = zero
            else:
                p2_v[p, pl.ds(col, 16)] = one

    def run_compute(in_buf, out_buf):
        @plsc.parallel_loop(0, _LGRP)
        def _(l):
            col = l * 16
            for p in _VALID_CELLS:
                g = _SRC[p]
                x = in_buf[g, pl.ds(col, 16)]
                out_buf[p, pl.ds(col, 16)] = jnp.where(x > 0.5, one, zero)
                out_buf[_NCELL + p, pl.ds(col, 16)] = jnp.where(
                    x < -0.5, one, zero)

    out_d = [[], []]
    p2_d = [[], []]
    for ci in range(nchunks):
        pb = ci & 1
        qb = (ci + 1) & 1
        if ci + 1 < nchunks:
            in_d[qb] = pltpu.async_copy(in_slice(ci + 1), ins[qb], semi[qb])
        in_d[pb].wait()
        for d in out_d[pb]:
            d.wait()
        for d in p2_d[pb]:
            d.wait()
        run_compute(ins[pb], outs[pb])
        bcol = base + ci * _BCHUNK
        out_d[pb] = [
            pltpu.async_copy(outs[pb].at[pl.ds(p0 * _PROJ_W, _PROJ_W)],
                             out_hbm.at[s, :, pl.ds(bcol, _BCHUNK)], semo[pb])
            for s, p0 in [(h, h) for h in range(_PROJ_H)]
            + [(12 + h, 12 + h) for h in range(_PROJ_H)]
        ]
        p2_d[pb] = [
            pltpu.async_copy(p2_v.at[pl.ds(h * _PROJ_W, _PROJ_W)],
                             out_hbm.at[24 + h, :, pl.ds(bcol, _BCHUNK)],
                             semp[pb])
            for h in range(_PROJ_H)
        ]

    for ds in out_d + p2_d:
        for d in ds:
            d.wait()


@jax.jit
def kernel(observation):
    if observation.ndim == 1:
        observation = observation[None, :]
    bsz = observation.shape[0]

    step = _NW * _BCHUNK
    bpad = ((bsz + step - 1) // step) * step
    obs_t = observation.astype(jnp.float32).T
    if bpad != bsz:
        obs_t = jnp.pad(obs_t, ((0, 0), (0, bpad - bsz)))
    nchunks = bpad // step

    run = pl.kernel(
        functools.partial(_sc_body, nchunks=nchunks),
        out_type=jax.ShapeDtypeStruct((_NSLAB, _PROJ_W, bpad), jnp.float32),
        mesh=plsc.VectorSubcoreMesh(core_axis_name="c", subcore_axis_name="s"),
        compiler_params=pltpu.CompilerParams(
            needs_layout_passes=False, use_tc_tiling_on_sc=True,
            skip_device_barrier=True, disable_bounds_checks=True,
            disable_semaphore_checks=True),
        scratch_types=[
            pltpu.VMEM((_N_ACTIONS, _BCHUNK), jnp.float32),
            pltpu.VMEM((_N_ACTIONS, _BCHUNK), jnp.float32),
            pltpu.VMEM((2 * _NCELL, _BCHUNK), jnp.float32),
            pltpu.VMEM((2 * _NCELL, _BCHUNK), jnp.float32),
            pltpu.VMEM((_NCELL, _BCHUNK), jnp.float32),
            pltpu.SemaphoreType.DMA,
            pltpu.SemaphoreType.DMA,
            pltpu.SemaphoreType.DMA,
            pltpu.SemaphoreType.DMA,
            pltpu.SemaphoreType.DMA,
            pltpu.SemaphoreType.DMA,
        ],
    )
    out3 = run(obs_t)
    board = jnp.transpose(out3.reshape(3, _PROJ_H, _PROJ_W, bpad),
                          (3, 0, 1, 2))[:bsz]
    return board.astype(observation.dtype)


# speedup vs baseline: 1.1925x; 1.0440x over previous
import functools

import jax
import jax.numpy as jnp
import numpy as np
from jax import lax
from jax.experimental import pallas as pl
from jax.experimental.pallas import tpu as pltpu
from jax.experimental.pallas import tpu_sc as plsc

_PROJ_H = 12
_PROJ_W = 12
_N_ACTIONS = 96
_NCELL = _PROJ_H * _PROJ_W
_ORIGINS = {0: (0, 4), 1: (4, 2), 2: (4, 6), 3: (8, 0), 4: (8, 4), 5: (8, 8)}


def _build_proj_index():
    idx = []
    for g in range(_N_ACTIONS):
        b = g // 16
        loc = g % 16
        r, c = (loc // 4, loc % 4)
        br, bc = _ORIGINS[b]
        idx.append((br + r) * _PROJ_W + (bc + c))
    return np.asarray(idx, dtype=np.int32)

_PROJ_IDX_NP = _build_proj_index()
_SRC = [None] * _NCELL
for _g, _p in enumerate(_PROJ_IDX_NP):
    _SRC[int(_p)] = _g
_VALID_CELLS = [p for p in range(_NCELL) if _SRC[p] is not None]

_NC = 2
_NS = 16
_NW = _NC * _NS
_BCHUNK = 128
_LGRP = _BCHUNK // 16
_NSLAB = 3 * _PROJ_H


def _sc_body(obs_hbm, out_hbm, in_v0, in_v1, out_v0, out_v1, p2_v,
             semi0, semi1, semo0, semo1, semp0, semp1, *, nchunks):
    wid = lax.axis_index("s") * _NC + lax.axis_index("c")
    base = wid * (nchunks * _BCHUNK)

    zero = jnp.zeros((16,), jnp.float32)
    one = jnp.ones((16,), jnp.float32)
    ins = [in_v0, in_v1]
    outs = [out_v0, out_v1]
    semi = [semi0, semi1]
    semo = [semo0, semo1]
    semp = [semp0, semp1]

    def in_slice(ci):
        return obs_hbm.at[:, pl.ds(base + ci * _BCHUNK, _BCHUNK)]

    in_d = [None, None]
    in_d[0] = pltpu.async_copy(in_slice(0), ins[0], semi[0])

    @plsc.parallel_loop(0, _LGRP)
    def _(l):
        col = l * 16
        for p in range(_NCELL):
            if _SRC[p] is None:
                for buf in outs:
                    buf[p, pl.ds(col, 16)] = zero
                    buf[_NCELL + p, pl.ds(col, 16)] = zero
                p2_v[p, pl.ds(col, 16)] = zero
            else:
                p2_v[p, pl.ds(col, 16)] = one

    def run_compute(in_buf, out_buf):
        @plsc.parallel_loop(0, _LGRP)
        def _(l):
            col = l * 16
            for p in _VALID_CELLS:
                g = _SRC[p]
                x = in_buf[g, pl.ds(col, 16)]
                out_buf[p, pl.ds(col, 16)] = jnp.where(x > 0.5, one, zero)
                out_buf[_NCELL + p, pl.ds(col, 16)] = jnp.where(
                    x < -0.5, one, zero)

    def issue_out(pb, bcol):
        for h in range(_PROJ_H):
            pltpu.async_copy(outs[pb].at[pl.ds(h * _PROJ_W, _PROJ_W)],
                             out_hbm.at[h, :, pl.ds(bcol, _BCHUNK)], semo[pb])
            pltpu.async_copy(outs[pb].at[pl.ds((12 + h) * _PROJ_W, _PROJ_W)],
                             out_hbm.at[12 + h, :, pl.ds(bcol, _BCHUNK)],
                             semo[pb])
            pltpu.async_copy(p2_v.at[pl.ds(h * _PROJ_W, _PROJ_W)],
                             out_hbm.at[24 + h, :, pl.ds(bcol, _BCHUNK)],
                             semp[pb])

    def wait_out(pb):
        for h in range(_PROJ_H):
            pltpu.make_async_copy(
                outs[pb].at[pl.ds(h * _PROJ_W, _PROJ_W)],
                out_hbm.at[h, :, pl.ds(0, _BCHUNK)], semo[pb]).wait()
            pltpu.make_async_copy(
                outs[pb].at[pl.ds((12 + h) * _PROJ_W, _PROJ_W)],
                out_hbm.at[12 + h, :, pl.ds(0, _BCHUNK)], semo[pb]).wait()
            pltpu.make_async_copy(
                p2_v.at[pl.ds(h * _PROJ_W, _PROJ_W)],
                out_hbm.at[24 + h, :, pl.ds(0, _BCHUNK)], semp[pb]).wait()

    def pair_body(pi, carry):
        bcol0 = base + 2 * pi * _BCHUNK
        for pb in (0, 1):
            bcol = bcol0 + pb * _BCHUNK
            nxt = bcol + _BCHUNK

            @pl.when(2 * pi + pb + 1 < nchunks)
            def _():
                pltpu.async_copy(
                    obs_hbm.at[:, pl.ds(nxt, _BCHUNK)], ins[1 - pb],
                    semi[1 - pb])

            pltpu.make_async_copy(
                obs_hbm.at[:, pl.ds(0, _BCHUNK)], ins[pb], semi[pb]).wait()

            @pl.when(2 * pi + pb >= 2)
            def _():
                wait_out(pb)

            run_compute(ins[pb], outs[pb])
            issue_out(pb, bcol)
        return carry

    lax.fori_loop(0, nchunks // 2, pair_body, 0)
    if nchunks % 2:
        ci = nchunks - 1
        pb = ci & 1
        pltpu.make_async_copy(
            obs_hbm.at[:, pl.ds(0, _BCHUNK)], ins[pb], semi[pb]).wait()
        if ci >= 2:
            wait_out(pb)
        run_compute(ins[pb], outs[pb])
        issue_out(pb, base + ci * _BCHUNK)
    if nchunks >= 2:
        wait_out(0)
        wait_out(1)
    else:
        wait_out(0)


@jax.jit
def kernel(observation):
    if observation.ndim == 1:
        observation = observation[None, :]
    bsz = observation.shape[0]

    step = _NW * _BCHUNK
    bpad = ((bsz + step - 1) // step) * step
    obs_t = observation.astype(jnp.float32).T
    if bpad != bsz:
        obs_t = jnp.pad(obs_t, ((0, 0), (0, bpad - bsz)))
    nchunks = bpad // step

    run = pl.kernel(
        functools.partial(_sc_body, nchunks=nchunks),
        out_type=jax.ShapeDtypeStruct((_NSLAB, _PROJ_W, bpad), jnp.float32),
        mesh=plsc.VectorSubcoreMesh(core_axis_name="c", subcore_axis_name="s"),
        compiler_params=pltpu.CompilerParams(
            needs_layout_passes=False, use_tc_tiling_on_sc=True,
            skip_device_barrier=True, disable_bounds_checks=True,
            disable_semaphore_checks=True),
        scratch_types=[
            pltpu.VMEM((_N_ACTIONS, _BCHUNK), jnp.float32),
            pltpu.VMEM((_N_ACTIONS, _BCHUNK), jnp.float32),
            pltpu.VMEM((2 * _NCELL, _BCHUNK), jnp.float32),
            pltpu.VMEM((2 * _NCELL, _BCHUNK), jnp.float32),
            pltpu.VMEM((_NCELL, _BCHUNK), jnp.float32),
            pltpu.SemaphoreType.DMA,
            pltpu.SemaphoreType.DMA,
            pltpu.SemaphoreType.DMA,
            pltpu.SemaphoreType.DMA,
            pltpu.SemaphoreType.DMA,
            pltpu.SemaphoreType.DMA,
        ],
    )
    out3 = run(obs_t)
    board = jnp.transpose(out3.reshape(3, _PROJ_H, _PROJ_W, bpad),
                          (3, 0, 1, 2))[:bsz]
    return board.astype(observation.dtype)


# speedup vs baseline: 1.2165x; 1.0201x over previous
import functools

import jax
import jax.numpy as jnp
import numpy as np
from jax import lax
from jax.experimental import pallas as pl
from jax.experimental.pallas import tpu as pltpu
from jax.experimental.pallas import tpu_sc as plsc

_PROJ_H = 12
_PROJ_W = 12
_N_ACTIONS = 96
_NCELL = _PROJ_H * _PROJ_W
_ORIGINS = {0: (0, 4), 1: (4, 2), 2: (4, 6), 3: (8, 0), 4: (8, 4), 5: (8, 8)}


def _build_proj_index():
    idx = []
    for g in range(_N_ACTIONS):
        b = g // 16
        loc = g % 16
        r, c = (loc // 4, loc % 4)
        br, bc = _ORIGINS[b]
        idx.append((br + r) * _PROJ_W + (bc + c))
    return np.asarray(idx, dtype=np.int32)

_PROJ_IDX_NP = _build_proj_index()
_SRC = [None] * _NCELL
for _g, _p in enumerate(_PROJ_IDX_NP):
    _SRC[int(_p)] = _g
_VALID_CELLS = [p for p in range(_NCELL) if _SRC[p] is not None]

_NC = 2
_NS = 16
_NW = _NC * _NS
_BCHUNK = 128
_LGRP = _BCHUNK // 16
_NSLAB = 3 * _PROJ_H


def _sc_body(obs_hbm, out_hbm, in_v0, in_v1, out_v0, out_v1, p2_v,
             semi0, semi1, semo0, semo1, semp0, semp1, *, nchunks):
    wid = lax.axis_index("s") * _NC + lax.axis_index("c")
    base = wid * (nchunks * _BCHUNK)

    zero = jnp.zeros((16,), jnp.float32)
    one = jnp.ones((16,), jnp.float32)
    ins = [in_v0, in_v1]
    outs = [out_v0, out_v1]
    semi = [semi0, semi1]
    semo = [semo0, semo1]
    semp = [semp0, semp1]

    def in_slice(ci):
        return obs_hbm.at[:, pl.ds(base + ci * _BCHUNK, _BCHUNK)]

    in_d = [None, None]
    in_d[0] = pltpu.async_copy(in_slice(0), ins[0], semi[0])

    @plsc.parallel_loop(0, _LGRP)
    def _(l):
        col = l * 16
        for p in range(_NCELL):
            if _SRC[p] is None:
                for buf in outs:
                    buf[p, pl.ds(col, 16)] = zero
                    buf[_NCELL + p, pl.ds(col, 16)] = zero
        for cls, h in enumerate((0, 4, 8)):
            for w in range(_PROJ_W):
                val = one if _SRC[h * _PROJ_W + w] is not None else zero
                p2_v[cls * _PROJ_W + w, pl.ds(col, 16)] = val

    def run_compute(in_buf, out_buf):
        @plsc.parallel_loop(0, _LGRP)
        def _(l):
            col = l * 16
            for p in _VALID_CELLS:
                g = _SRC[p]
                x = in_buf[g, pl.ds(col, 16)]
                out_buf[p, pl.ds(col, 16)] = jnp.where(x > 0.5, one, zero)
                out_buf[_NCELL + p, pl.ds(col, 16)] = jnp.where(
                    x < -0.5, one, zero)

    def issue_out(pb, bcol):
        for h in range(_PROJ_H):
            pltpu.async_copy(outs[pb].at[pl.ds(h * _PROJ_W, _PROJ_W)],
                             out_hbm.at[h, :, pl.ds(bcol, _BCHUNK)], semo[pb])
            pltpu.async_copy(outs[pb].at[pl.ds((12 + h) * _PROJ_W, _PROJ_W)],
                             out_hbm.at[12 + h, :, pl.ds(bcol, _BCHUNK)],
                             semo[pb])
            pltpu.async_copy(p2_v.at[pl.ds((h // 4) * _PROJ_W, _PROJ_W)],
                             out_hbm.at[24 + h, :, pl.ds(bcol, _BCHUNK)],
                             semp[pb])

    def wait_out(pb):
        for h in range(_PROJ_H):
            pltpu.make_async_copy(
                outs[pb].at[pl.ds(h * _PROJ_W, _PROJ_W)],
                out_hbm.at[h, :, pl.ds(0, _BCHUNK)], semo[pb]).wait()
            pltpu.make_async_copy(
                outs[pb].at[pl.ds((12 + h) * _PROJ_W, _PROJ_W)],
                out_hbm.at[12 + h, :, pl.ds(0, _BCHUNK)], semo[pb]).wait()
            pltpu.make_async_copy(
                p2_v.at[pl.ds((h // 4) * _PROJ_W, _PROJ_W)],
                out_hbm.at[24 + h, :, pl.ds(0, _BCHUNK)], semp[pb]).wait()

    def pair_body(pi, carry):
        bcol0 = base + 2 * pi * _BCHUNK
        for pb in (0, 1):
            bcol = bcol0 + pb * _BCHUNK
            nxt = bcol + _BCHUNK

            @pl.when(2 * pi + pb + 1 < nchunks)
            def _():
                pltpu.async_copy(
                    obs_hbm.at[:, pl.ds(nxt, _BCHUNK)], ins[1 - pb],
                    semi[1 - pb])

            pltpu.make_async_copy(
                obs_hbm.at[:, pl.ds(0, _BCHUNK)], ins[pb], semi[pb]).wait()

            @pl.when(2 * pi + pb >= 2)
            def _():
                wait_out(pb)

            run_compute(ins[pb], outs[pb])
            issue_out(pb, bcol)
        return carry

    lax.fori_loop(0, nchunks // 2, pair_body, 0)
    if nchunks % 2:
        ci = nchunks - 1
        pb = ci & 1
        pltpu.make_async_copy(
            obs_hbm.at[:, pl.ds(0, _BCHUNK)], ins[pb], semi[pb]).wait()
        if ci >= 2:
            wait_out(pb)
        run_compute(ins[pb], outs[pb])
        issue_out(pb, base + ci * _BCHUNK)
    if nchunks >= 2:
        wait_out(0)
        wait_out(1)
    else:
        wait_out(0)


@jax.jit
def kernel(observation):
    if observation.ndim == 1:
        observation = observation[None, :]
    bsz = observation.shape[0]

    step = _NW * _BCHUNK
    bpad = ((bsz + step - 1) // step) * step
    obs_t = observation.astype(jnp.float32).T
    if bpad != bsz:
        obs_t = jnp.pad(obs_t, ((0, 0), (0, bpad - bsz)))
    nchunks = bpad // step

    run = pl.kernel(
        functools.partial(_sc_body, nchunks=nchunks),
        out_type=jax.ShapeDtypeStruct((_NSLAB, _PROJ_W, bpad), jnp.float32),
        mesh=plsc.VectorSubcoreMesh(core_axis_name="c", subcore_axis_name="s"),
        compiler_params=pltpu.CompilerParams(
            needs_layout_passes=False, use_tc_tiling_on_sc=True,
            skip_device_barrier=True, disable_bounds_checks=True,
            disable_semaphore_checks=True),
        scratch_types=[
            pltpu.VMEM((_N_ACTIONS, _BCHUNK), jnp.float32),
            pltpu.VMEM((_N_ACTIONS, _BCHUNK), jnp.float32),
            pltpu.VMEM((2 * _NCELL, _BCHUNK), jnp.float32),
            pltpu.VMEM((2 * _NCELL, _BCHUNK), jnp.float32),
            pltpu.VMEM((3 * _PROJ_W, _BCHUNK), jnp.float32),
            pltpu.SemaphoreType.DMA,
            pltpu.SemaphoreType.DMA,
            pltpu.SemaphoreType.DMA,
            pltpu.SemaphoreType.DMA,
            pltpu.SemaphoreType.DMA,
            pltpu.SemaphoreType.DMA,
        ],
    )
    out3 = run(obs_t)
    board = jnp.transpose(out3.reshape(3, _PROJ_H, _PROJ_W, bpad),
                          (3, 0, 1, 2))[:bsz]
    return board.astype(observation.dtype)


# speedup vs baseline: 1.3656x; 1.1225x over previous
import functools

import jax
import jax.numpy as jnp
import numpy as np
from jax import lax
from jax.experimental import pallas as pl
from jax.experimental.pallas import tpu as pltpu
from jax.experimental.pallas import tpu_sc as plsc

_PROJ_H = 12
_PROJ_W = 12
_N_ACTIONS = 96
_NCELL = _PROJ_H * _PROJ_W
_ORIGINS = {0: (0, 4), 1: (4, 2), 2: (4, 6), 3: (8, 0), 4: (8, 4), 5: (8, 8)}


def _build_proj_index():
    idx = []
    for g in range(_N_ACTIONS):
        b = g // 16
        loc = g % 16
        r, c = (loc // 4, loc % 4)
        br, bc = _ORIGINS[b]
        idx.append((br + r) * _PROJ_W + (bc + c))
    return np.asarray(idx, dtype=np.int32)

_PROJ_IDX_NP = _build_proj_index()
_SRC = [None] * _NCELL
for _g, _p in enumerate(_PROJ_IDX_NP):
    _SRC[int(_p)] = _g
_VALID_CELLS = [p for p in range(_NCELL) if _SRC[p] is not None]

_NC = 2
_NS = 16
_NW = _NC * _NS
_BCHUNK = 128
_LGRP = _BCHUNK // 16
_NSLAB = 3 * _PROJ_H


def _sc_body(obs_hbm, out_hbm, in_v0, in_v1, out_v0, out_v1, p2_v,
             semi0, semi1, semo0, semo1, semp0, semp1, *, nchunks):
    wid = lax.axis_index("s") * _NC + lax.axis_index("c")
    base = wid * (nchunks * _BCHUNK)

    zero = jnp.zeros((16,), jnp.float32)
    one = jnp.ones((16,), jnp.float32)
    ins = [in_v0, in_v1]
    outs = [out_v0, out_v1]
    semi = [semi0, semi1]
    semo = [semo0, semo1]
    semp = [semp0, semp1]

    def in_slice(ci):
        return obs_hbm.at[:, pl.ds(base + ci * _BCHUNK, _BCHUNK)]

    in_d = [None, None]
    in_d[0] = pltpu.async_copy(in_slice(0), ins[0], semi[0])

    @plsc.parallel_loop(0, _LGRP)
    def _(l):
        col = l * 16
        for p in range(_NCELL):
            if _SRC[p] is None:
                for buf in outs:
                    buf[p, pl.ds(col, 16)] = zero
                    buf[_NCELL + p, pl.ds(col, 16)] = zero
        for cls, h in enumerate((0, 4, 8)):
            for w in range(_PROJ_W):
                val = one if _SRC[h * _PROJ_W + w] is not None else zero
                p2_v[cls * _PROJ_W + w, pl.ds(col, 16)] = val

    def run_compute(in_buf, out_buf, cells):
        @plsc.parallel_loop(0, _LGRP)
        def _(l):
            col = l * 16
            for p in cells:
                g = _SRC[p]
                x = in_buf[g, pl.ds(col, 16)]
                out_buf[p, pl.ds(col, 16)] = jnp.where(x > 0.5, one, zero)
                out_buf[_NCELL + p, pl.ds(col, 16)] = jnp.where(
                    x < -0.5, one, zero)

    cells_lo = [p for p in _VALID_CELLS if p // _PROJ_W < 8]
    cells_hi = [p for p in _VALID_CELLS if p // _PROJ_W >= 8]

    def issue_out(pb, bcol, h_range):
        for h in h_range:
            pltpu.async_copy(outs[pb].at[pl.ds(h * _PROJ_W, _PROJ_W)],
                             out_hbm.at[h, :, pl.ds(bcol, _BCHUNK)], semo[pb])
            pltpu.async_copy(outs[pb].at[pl.ds((12 + h) * _PROJ_W, _PROJ_W)],
                             out_hbm.at[12 + h, :, pl.ds(bcol, _BCHUNK)],
                             semo[pb])
            pltpu.async_copy(p2_v.at[pl.ds((h // 4) * _PROJ_W, _PROJ_W)],
                             out_hbm.at[24 + h, :, pl.ds(bcol, _BCHUNK)],
                             semp[pb])

    def wait_out(pb):
        for h in range(_PROJ_H):
            pltpu.make_async_copy(
                outs[pb].at[pl.ds(h * _PROJ_W, _PROJ_W)],
                out_hbm.at[h, :, pl.ds(0, _BCHUNK)], semo[pb]).wait()
            pltpu.make_async_copy(
                outs[pb].at[pl.ds((12 + h) * _PROJ_W, _PROJ_W)],
                out_hbm.at[12 + h, :, pl.ds(0, _BCHUNK)], semo[pb]).wait()
            pltpu.make_async_copy(
                p2_v.at[pl.ds((h // 4) * _PROJ_W, _PROJ_W)],
                out_hbm.at[24 + h, :, pl.ds(0, _BCHUNK)], semp[pb]).wait()

    def pair_body(pi, carry):
        bcol0 = base + 2 * pi * _BCHUNK
        for pb in (0, 1):
            bcol = bcol0 + pb * _BCHUNK
            nxt = bcol + _BCHUNK

            @pl.when(2 * pi + pb + 1 < nchunks)
            def _():
                pltpu.async_copy(
                    obs_hbm.at[:, pl.ds(nxt, _BCHUNK)], ins[1 - pb],
                    semi[1 - pb])

            pltpu.make_async_copy(
                obs_hbm.at[:, pl.ds(0, _BCHUNK)], ins[pb], semi[pb]).wait()

            @pl.when(2 * pi + pb >= 2)
            def _():
                wait_out(pb)

            run_compute(ins[pb], outs[pb], cells_lo)
            issue_out(pb, bcol, range(8))
            run_compute(ins[pb], outs[pb], cells_hi)
            issue_out(pb, bcol, range(8, _PROJ_H))
        return carry

    lax.fori_loop(0, nchunks // 2, pair_body, 0)
    if nchunks % 2:
        ci = nchunks - 1
        pb = ci & 1
        pltpu.make_async_copy(
            obs_hbm.at[:, pl.ds(0, _BCHUNK)], ins[pb], semi[pb]).wait()
        if ci >= 2:
            wait_out(pb)
        run_compute(ins[pb], outs[pb], cells_lo)
        issue_out(pb, base + ci * _BCHUNK, range(8))
        run_compute(ins[pb], outs[pb], cells_hi)
        issue_out(pb, base + ci * _BCHUNK, range(8, _PROJ_H))
    if nchunks >= 2:
        wait_out(0)
        wait_out(1)
    else:
        wait_out(0)


@jax.jit
def kernel(observation):
    if observation.ndim == 1:
        observation = observation[None, :]
    bsz = observation.shape[0]

    step = _NW * _BCHUNK
    bpad = ((bsz + step - 1) // step) * step
    obs_t = observation.astype(jnp.float32).T
    if bpad != bsz:
        obs_t = jnp.pad(obs_t, ((0, 0), (0, bpad - bsz)))
    nchunks = bpad // step

    run = pl.kernel(
        functools.partial(_sc_body, nchunks=nchunks),
        out_type=jax.ShapeDtypeStruct((_NSLAB, _PROJ_W, bpad), jnp.float32),
        mesh=plsc.VectorSubcoreMesh(core_axis_name="c", subcore_axis_name="s"),
        compiler_params=pltpu.CompilerParams(
            needs_layout_passes=False, use_tc_tiling_on_sc=True,
            skip_device_barrier=True, disable_bounds_checks=True,
            disable_semaphore_checks=True),
        scratch_types=[
            pltpu.VMEM((_N_ACTIONS, _BCHUNK), jnp.float32),
            pltpu.VMEM((_N_ACTIONS, _BCHUNK), jnp.float32),
            pltpu.VMEM((2 * _NCELL, _BCHUNK), jnp.float32),
            pltpu.VMEM((2 * _NCELL, _BCHUNK), jnp.float32),
            pltpu.VMEM((3 * _PROJ_W, _BCHUNK), jnp.float32),
            pltpu.SemaphoreType.DMA,
            pltpu.SemaphoreType.DMA,
            pltpu.SemaphoreType.DMA,
            pltpu.SemaphoreType.DMA,
            pltpu.SemaphoreType.DMA,
            pltpu.SemaphoreType.DMA,
        ],
    )
    out3 = run(obs_t)
    board = jnp.transpose(out3.reshape(3, _PROJ_H, _PROJ_W, bpad),
                          (3, 0, 1, 2))[:bsz]
    return board.astype(observation.dtype)


# speedup vs baseline: 1.4125x; 1.0344x over previous
import functools

import jax
import jax.numpy as jnp
import numpy as np
from jax import lax
from jax.experimental import pallas as pl
from jax.experimental.pallas import tpu as pltpu
from jax.experimental.pallas import tpu_sc as plsc

_PROJ_H = 12
_PROJ_W = 12
_N_ACTIONS = 96
_NCELL = _PROJ_H * _PROJ_W
_ORIGINS = {0: (0, 4), 1: (4, 2), 2: (4, 6), 3: (8, 0), 4: (8, 4), 5: (8, 8)}


def _build_proj_index():
    idx = []
    for g in range(_N_ACTIONS):
        b = g // 16
        loc = g % 16
        r, c = (loc // 4, loc % 4)
        br, bc = _ORIGINS[b]
        idx.append((br + r) * _PROJ_W + (bc + c))
    return np.asarray(idx, dtype=np.int32)

_PROJ_IDX_NP = _build_proj_index()
_SRC = [None] * _NCELL
for _g, _p in enumerate(_PROJ_IDX_NP):
    _SRC[int(_p)] = _g
_VALID_CELLS = [p for p in range(_NCELL) if _SRC[p] is not None]

_NC = 2
_NS = 16
_NW = _NC * _NS
_BCHUNK = 128
_LGRP = _BCHUNK // 16
_NSLAB = 3 * _PROJ_H


def _sc_body(obs_hbm, out_hbm, in_v0, in_v1, out_v0, out_v1, p2_v,
             semi0, semi1, semo0, semo1, semp0, semp1, *, nchunks):
    wid = lax.axis_index("s") * _NC + lax.axis_index("c")
    base = wid * (nchunks * _BCHUNK)

    zero = jnp.zeros((16,), jnp.float32)
    one = jnp.ones((16,), jnp.float32)
    ins = [in_v0, in_v1]
    outs = [out_v0, out_v1]
    semi = [semi0, semi1]
    semo = [semo0, semo1]
    semp = [semp0, semp1]

    def in_slice(ci):
        return obs_hbm.at[:, pl.ds(base + ci * _BCHUNK, _BCHUNK)]

    in_d = [None, None]
    in_d[0] = pltpu.async_copy(in_slice(0), ins[0], semi[0])

    @plsc.parallel_loop(0, _LGRP)
    def _(l):
        col = l * 16
        for p in range(_NCELL):
            if _SRC[p] is None:
                for buf in outs:
                    buf[p, pl.ds(col, 16)] = zero
                    buf[_NCELL + p, pl.ds(col, 16)] = zero
        for cls, h in enumerate((0, 4, 8)):
            for w in range(_PROJ_W):
                val = one if _SRC[h * _PROJ_W + w] is not None else zero
                p2_v[cls * _PROJ_W + w, pl.ds(col, 16)] = val

    def run_compute(in_buf, out_buf, cells):
        @plsc.parallel_loop(0, _LGRP)
        def _(l):
            col = l * 16
            for p in cells:
                g = _SRC[p]
                x = in_buf[g, pl.ds(col, 16)]
                out_buf[p, pl.ds(col, 16)] = jnp.where(x > 0.5, one, zero)
                out_buf[_NCELL + p, pl.ds(col, 16)] = jnp.where(
                    x < -0.5, one, zero)

    _SEGS = [(0, 4), (4, 8), (8, 10), (10, 12)]
    seg_cells = [[p for p in _VALID_CELLS if a <= p // _PROJ_W < b]
                 for a, b in _SEGS]

    def issue_out(pb, bcol, h_range):
        for h in h_range:
            pltpu.async_copy(outs[pb].at[pl.ds(h * _PROJ_W, _PROJ_W)],
                             out_hbm.at[h, :, pl.ds(bcol, _BCHUNK)], semo[pb])
            pltpu.async_copy(outs[pb].at[pl.ds((12 + h) * _PROJ_W, _PROJ_W)],
                             out_hbm.at[12 + h, :, pl.ds(bcol, _BCHUNK)],
                             semo[pb])
            pltpu.async_copy(p2_v.at[pl.ds((h // 4) * _PROJ_W, _PROJ_W)],
                             out_hbm.at[24 + h, :, pl.ds(bcol, _BCHUNK)],
                             semp[pb])

    def wait_out(pb):
        for h in range(_PROJ_H):
            pltpu.make_async_copy(
                outs[pb].at[pl.ds(h * _PROJ_W, _PROJ_W)],
                out_hbm.at[h, :, pl.ds(0, _BCHUNK)], semo[pb]).wait()
            pltpu.make_async_copy(
                outs[pb].at[pl.ds((12 + h) * _PROJ_W, _PROJ_W)],
                out_hbm.at[12 + h, :, pl.ds(0, _BCHUNK)], semo[pb]).wait()
            pltpu.make_async_copy(
                p2_v.at[pl.ds((h // 4) * _PROJ_W, _PROJ_W)],
                out_hbm.at[24 + h, :, pl.ds(0, _BCHUNK)], semp[pb]).wait()

    def pair_body(pi, carry):
        bcol0 = base + 2 * pi * _BCHUNK
        for pb in (0, 1):
            bcol = bcol0 + pb * _BCHUNK
            nxt = bcol + _BCHUNK

            @pl.when(2 * pi + pb + 1 < nchunks)
            def _():
                pltpu.async_copy(
                    obs_hbm.at[:, pl.ds(nxt, _BCHUNK)], ins[1 - pb],
                    semi[1 - pb])

            pltpu.make_async_copy(
                obs_hbm.at[:, pl.ds(0, _BCHUNK)], ins[pb], semi[pb]).wait()

            @pl.when(2 * pi + pb >= 2)
            def _():
                wait_out(pb)

            for (a, b), cl in zip(_SEGS, seg_cells):
                run_compute(ins[pb], outs[pb], cl)
                issue_out(pb, bcol, range(a, b))
        return carry

    lax.fori_loop(0, nchunks // 2, pair_body, 0)
    if nchunks % 2:
        ci = nchunks - 1
        pb = ci & 1
        pltpu.make_async_copy(
            obs_hbm.at[:, pl.ds(0, _BCHUNK)], ins[pb], semi[pb]).wait()
        if ci >= 2:
            wait_out(pb)
        for (a, b), cl in zip(_SEGS, seg_cells):
            run_compute(ins[pb], outs[pb], cl)
            issue_out(pb, base + ci * _BCHUNK, range(a, b))
    if nchunks >= 2:
        wait_out(0)
        wait_out(1)
    else:
        wait_out(0)


@jax.jit
def kernel(observation):
    if observation.ndim == 1:
        observation = observation[None, :]
    bsz = observation.shape[0]

    step = _NW * _BCHUNK
    bpad = ((bsz + step - 1) // step) * step
    obs_t = observation.astype(jnp.float32).T
    if bpad != bsz:
        obs_t = jnp.pad(obs_t, ((0, 0), (0, bpad - bsz)))
    nchunks = bpad // step

    run = pl.kernel(
        functools.partial(_sc_body, nchunks=nchunks),
        out_type=jax.ShapeDtypeStruct((_NSLAB, _PROJ_W, bpad), jnp.float32),
        mesh=plsc.VectorSubcoreMesh(core_axis_name="c", subcore_axis_name="s"),
        compiler_params=pltpu.CompilerParams(
            needs_layout_passes=False, use_tc_tiling_on_sc=True,
            skip_device_barrier=True, disable_bounds_checks=True,
            disable_semaphore_checks=True),
        scratch_types=[
            pltpu.VMEM((_N_ACTIONS, _BCHUNK), jnp.float32),
            pltpu.VMEM((_N_ACTIONS, _BCHUNK), jnp.float32),
            pltpu.VMEM((2 * _NCELL, _BCHUNK), jnp.float32),
            pltpu.VMEM((2 * _NCELL, _BCHUNK), jnp.float32),
            pltpu.VMEM((3 * _PROJ_W, _BCHUNK), jnp.float32),
            pltpu.SemaphoreType.DMA,
            pltpu.SemaphoreType.DMA,
            pltpu.SemaphoreType.DMA,
            pltpu.SemaphoreType.DMA,
            pltpu.SemaphoreType.DMA,
            pltpu.SemaphoreType.DMA,
        ],
    )
    out3 = run(obs_t)
    board = jnp.transpose(out3.reshape(3, _PROJ_H, _PROJ_W, bpad),
                          (3, 0, 1, 2))[:bsz]
    return board.astype(observation.dtype)


# speedup vs baseline: 1.4184x; 1.0042x over previous
import functools

import jax
import jax.numpy as jnp
import numpy as np
from jax import lax
from jax.experimental import pallas as pl
from jax.experimental.pallas import tpu as pltpu
from jax.experimental.pallas import tpu_sc as plsc

_PROJ_H = 12
_PROJ_W = 12
_N_ACTIONS = 96
_NCELL = _PROJ_H * _PROJ_W
_ORIGINS = {0: (0, 4), 1: (4, 2), 2: (4, 6), 3: (8, 0), 4: (8, 4), 5: (8, 8)}


def _build_proj_index():
    idx = []
    for g in range(_N_ACTIONS):
        b = g // 16
        loc = g % 16
        r, c = (loc // 4, loc % 4)
        br, bc = _ORIGINS[b]
        idx.append((br + r) * _PROJ_W + (bc + c))
    return np.asarray(idx, dtype=np.int32)

_PROJ_IDX_NP = _build_proj_index()
_SRC = [None] * _NCELL
for _g, _p in enumerate(_PROJ_IDX_NP):
    _SRC[int(_p)] = _g
_VALID_CELLS = [p for p in range(_NCELL) if _SRC[p] is not None]

_NC = 2
_NS = 16
_NW = _NC * _NS
_BCHUNK = 128
_LGRP = _BCHUNK // 16
_NSLAB = 3 * _PROJ_H


def _sc_body(obs_hbm, out_hbm, in_v0, in_v1, out_v0, out_v1, p2_v,
             semi0, semi1, semo0, semo1, semp0, semp1, *, nchunks):
    wid = lax.axis_index("s") * _NC + lax.axis_index("c")
    base = wid * (nchunks * _BCHUNK)

    zero = jnp.zeros((16,), jnp.float32)
    one = jnp.ones((16,), jnp.float32)
    ins = [in_v0, in_v1]
    outs = [out_v0, out_v1]
    semi = [semi0, semi1]
    semo = [semo0, semo1]
    semp = [semp0, semp1]

    def in_slice(ci):
        return obs_hbm.at[:, pl.ds(base + ci * _BCHUNK, _BCHUNK)]

    in_d = [None, None]
    in_d[0] = pltpu.async_copy(in_slice(0), ins[0], semi[0])

    @plsc.parallel_loop(0, _LGRP)
    def _(l):
        col = l * 16
        for p in range(_NCELL):
            if _SRC[p] is None:
                for buf in outs:
                    buf[p, pl.ds(col, 16)] = zero
                    buf[_NCELL + p, pl.ds(col, 16)] = zero
        for cls, h in enumerate((0, 4, 8)):
            for w in range(_PROJ_W):
                val = one if _SRC[h * _PROJ_W + w] is not None else zero
                p2_v[cls * _PROJ_W + w, pl.ds(col, 16)] = val

    def run_compute(in_buf, out_buf, cells):
        @plsc.parallel_loop(0, _LGRP)
        def _(l):
            col = l * 16
            for p in cells:
                g = _SRC[p]
                x = in_buf[g, pl.ds(col, 16)]
                out_buf[p, pl.ds(col, 16)] = jnp.where(x > 0.5, one, zero)
                out_buf[_NCELL + p, pl.ds(col, 16)] = jnp.where(
                    x < -0.5, one, zero)

    _SEGS = [(0, 2), (2, 4), (4, 6), (6, 8), (8, 10), (10, 12)]
    seg_cells = [[p for p in _VALID_CELLS if a <= p // _PROJ_W < b]
                 for a, b in _SEGS]

    def issue_out(pb, bcol, h_range):
        for h in h_range:
            pltpu.async_copy(outs[pb].at[pl.ds(h * _PROJ_W, _PROJ_W)],
                             out_hbm.at[h, :, pl.ds(bcol, _BCHUNK)], semo[pb])
            pltpu.async_copy(outs[pb].at[pl.ds((12 + h) * _PROJ_W, _PROJ_W)],
                             out_hbm.at[12 + h, :, pl.ds(bcol, _BCHUNK)],
                             semo[pb])
            pltpu.async_copy(p2_v.at[pl.ds((h // 4) * _PROJ_W, _PROJ_W)],
                             out_hbm.at[24 + h, :, pl.ds(bcol, _BCHUNK)],
                             semp[pb])

    def wait_out(pb):
        for h in range(_PROJ_H):
            pltpu.make_async_copy(
                outs[pb].at[pl.ds(h * _PROJ_W, _PROJ_W)],
                out_hbm.at[h, :, pl.ds(0, _BCHUNK)], semo[pb]).wait()
            pltpu.make_async_copy(
                outs[pb].at[pl.ds((12 + h) * _PROJ_W, _PROJ_W)],
                out_hbm.at[12 + h, :, pl.ds(0, _BCHUNK)], semo[pb]).wait()
            pltpu.make_async_copy(
                p2_v.at[pl.ds((h // 4) * _PROJ_W, _PROJ_W)],
                out_hbm.at[24 + h, :, pl.ds(0, _BCHUNK)], semp[pb]).wait()

    def pair_body(pi, carry):
        bcol0 = base + 2 * pi * _BCHUNK
        for pb in (0, 1):
            bcol = bcol0 + pb * _BCHUNK
            nxt = bcol + _BCHUNK

            @pl.when(2 * pi + pb + 1 < nchunks)
            def _():
                pltpu.async_copy(
                    obs_hbm.at[:, pl.ds(nxt, _BCHUNK)], ins[1 - pb],
                    semi[1 - pb])

            pltpu.make_async_copy(
                obs_hbm.at[:, pl.ds(0, _BCHUNK)], ins[pb], semi[pb]).wait()

            @pl.when(2 * pi + pb >= 2)
            def _():
                wait_out(pb)

            for (a, b), cl in zip(_SEGS, seg_cells):
                run_compute(ins[pb], outs[pb], cl)
                issue_out(pb, bcol, range(a, b))
        return carry

    lax.fori_loop(0, nchunks // 2, pair_body, 0)
    if nchunks % 2:
        ci = nchunks - 1
        pb = ci & 1
        pltpu.make_async_copy(
            obs_hbm.at[:, pl.ds(0, _BCHUNK)], ins[pb], semi[pb]).wait()
        if ci >= 2:
            wait_out(pb)
        for (a, b), cl in zip(_SEGS, seg_cells):
            run_compute(ins[pb], outs[pb], cl)
            issue_out(pb, base + ci * _BCHUNK, range(a, b))
    if nchunks >= 2:
        wait_out(0)
        wait_out(1)
    else:
        wait_out(0)


@jax.jit
def kernel(observation):
    if observation.ndim == 1:
        observation = observation[None, :]
    bsz = observation.shape[0]

    step = _NW * _BCHUNK
    bpad = ((bsz + step - 1) // step) * step
    obs_t = observation.astype(jnp.float32).T
    if bpad != bsz:
        obs_t = jnp.pad(obs_t, ((0, 0), (0, bpad - bsz)))
    nchunks = bpad // step

    run = pl.kernel(
        functools.partial(_sc_body, nchunks=nchunks),
        out_type=jax.ShapeDtypeStruct((_NSLAB, _PROJ_W, bpad), jnp.float32),
        mesh=plsc.VectorSubcoreMesh(core_axis_name="c", subcore_axis_name="s"),
        compiler_params=pltpu.CompilerParams(
            needs_layout_passes=False, use_tc_tiling_on_sc=True,
            skip_device_barrier=True, disable_bounds_checks=True,
            disable_semaphore_checks=True),
        scratch_types=[
            pltpu.VMEM((_N_ACTIONS, _BCHUNK), jnp.float32),
            pltpu.VMEM((_N_ACTIONS, _BCHUNK), jnp.float32),
            pltpu.VMEM((2 * _NCELL, _BCHUNK), jnp.float32),
            pltpu.VMEM((2 * _NCELL, _BCHUNK), jnp.float32),
            pltpu.VMEM((3 * _PROJ_W, _BCHUNK), jnp.float32),
            pltpu.SemaphoreType.DMA,
            pltpu.SemaphoreType.DMA,
            pltpu.SemaphoreType.DMA,
            pltpu.SemaphoreType.DMA,
            pltpu.SemaphoreType.DMA,
            pltpu.SemaphoreType.DMA,
        ],
    )
    out3 = run(obs_t)
    board = jnp.transpose(out3.reshape(3, _PROJ_H, _PROJ_W, bpad),
                          (3, 0, 1, 2))[:bsz]
    return board.astype(observation.dtype)
